# baseline (speedup 1.0000x reference)
import functools

import jax
import jax.numpy as jnp
from jax import lax
from jax.experimental import pallas as pl
from jax.experimental.pallas import tpu as pltpu
from jax.experimental.pallas import tpu_sc as plsc

_TOP_K = 2
_TILE = 4096
_N = 32768
_E = 64
_NC = 2
_NS = 16
_L = 16
_NW = _NC * _NS
_TPW = _N // _NW


def _logits_kernel(x_ref, w_ref, lt_ref):
    x = x_ref[...]
    w = w_ref[...]
    logits = jax.lax.dot_general(
        x, w, (((1,), (1,)), ((), ())),
        preferred_element_type=jnp.float32)
    lt_ref[...] = logits.T


def _compute_logits_t(x, W):
    n, d = x.shape
    e = W.shape[0]
    return pl.pallas_call(
        _logits_kernel,
        grid=(n // _TILE,),
        in_specs=[
            pl.BlockSpec((_TILE, d), lambda i: (i, 0)),
            pl.BlockSpec((e, d), lambda i: (0, 0)),
        ],
        out_specs=pl.BlockSpec((e, _TILE), lambda i: (0, i)),
        out_shape=jax.ShapeDtypeStruct((e, n), jnp.float32),
        compiler_params=pltpu.CompilerParams(
            dimension_semantics=("parallel",)),
    )(x, W)


@functools.partial(
    pl.kernel,
    out_type=[
        jax.ShapeDtypeStruct((_N,), jnp.int32),
        jax.ShapeDtypeStruct((_N,), jnp.int32),
        jax.ShapeDtypeStruct((_N,), jnp.float32),
        jax.ShapeDtypeStruct((_N,), jnp.float32),
    ],
    mesh=plsc.VectorSubcoreMesh(
        core_axis_name="c", subcore_axis_name="s",
        num_cores=_NC, num_subcores=_NS),
    scratch_types=[
        pltpu.VMEM((_E, _TPW), jnp.float32),
        pltpu.VMEM((_TPW,), jnp.int32),
        pltpu.VMEM((_TPW,), jnp.int32),
        pltpu.VMEM((_TPW,), jnp.float32),
        pltpu.VMEM((_TPW,), jnp.float32),
    ],
)
def _sc_top2(lt_hbm, i1_hbm, i2_hbm, g1_hbm, g2_hbm,
             lt_v, i1_v, i2_v, g1_v, g2_v):
    wid = lax.axis_index("s") * _NC + lax.axis_index("c")
    base = wid * _TPW
    pltpu.sync_copy(lt_hbm.at[:, pl.ds(base, _TPW)], lt_v)

    def group(g, carry):
        sl = pl.ds(g * _L, _L)
        m1 = lt_v[0, sl]
        i1 = jnp.zeros((_L,), jnp.int32)
        m2 = jnp.full((_L,), -jnp.inf, jnp.float32)
        i2 = jnp.zeros((_L,), jnp.int32)

        def body(e, c):
            m1, i1, m2, i2 = c
            v = lt_v[e, sl]
            e_vec = jnp.full((_L,), 0, jnp.int32) + e
            gt1 = v > m1
            gt2 = v > m2
            m2n = jnp.where(gt1, m1, jnp.where(gt2, v, m2))
            i2n = jnp.where(gt1, i1, jnp.where(gt2, e_vec, i2))
            m1n = jnp.where(gt1, v, m1)
            i1n = jnp.where(gt1, e_vec, i1)
            return m1n, i1n, m2n, i2n

        m1, i1, m2, i2 = lax.fori_loop(1, _E, body, (m1, i1, m2, i2))
        t = jnp.exp(m2 - m1)
        i1_v[sl] = i1
        i2_v[sl] = i2
        g1_v[sl] = 1.0 / (1.0 + t)
        g2_v[sl] = t / (1.0 + t)
        return carry

    lax.fori_loop(0, _TPW // _L, group, 0)
    pltpu.sync_copy(i1_v, i1_hbm.at[pl.ds(base, _TPW)])
    pltpu.sync_copy(i2_v, i2_hbm.at[pl.ds(base, _TPW)])
    pltpu.sync_copy(g1_v, g1_hbm.at[pl.ds(base, _TPW)])
    pltpu.sync_copy(g2_v, g2_hbm.at[pl.ds(base, _TPW)])


@jax.jit
def kernel(x, W):
    lt = _compute_logits_t(x, W)
    i1, i2, g1, g2 = _sc_top2(lt)
    idx = jnp.stack([i1, i2], axis=1)
    gates = jnp.stack([g1, g2], axis=1)
    return idx, gates


# speedup vs baseline: 1.1728x; 1.1728x over previous
import jax
import jax.numpy as jnp
from jax.experimental import pallas as pl
from jax.experimental.pallas import tpu as pltpu

_TOP_K = 2
_TILE = 4096


def _router_kernel(x_ref, w_ref, idx_ref, gate_ref):
    x = x_ref[...]
    w = w_ref[...]
    logits = jax.lax.dot_general(
        x, w, (((1,), (1,)), ((), ())),
        preferred_element_type=jnp.float32)

    m1 = jnp.max(logits, axis=1)
    i1 = jnp.argmax(logits, axis=1).astype(jnp.int32)
    col = jax.lax.broadcasted_iota(jnp.int32, logits.shape, 1)
    masked = jnp.where(col == i1[:, None], -jnp.inf, logits)
    m2 = jnp.max(masked, axis=1)
    i2 = jnp.argmax(masked, axis=1).astype(jnp.int32)

    t = jnp.exp(m2 - m1)
    g1 = 1.0 / (1.0 + t)
    g2 = t / (1.0 + t)

    idx_ref[...] = jnp.stack([i1, i2], axis=1)
    gate_ref[...] = jnp.stack([g1, g2], axis=1)


@jax.jit
def kernel(x, W):
    n, d = x.shape
    e = W.shape[0]
    grid = (n // _TILE,)
    idx, gates = pl.pallas_call(
        _router_kernel,
        grid=grid,
        in_specs=[
            pl.BlockSpec((_TILE, d), lambda i: (i, 0)),
            pl.BlockSpec((e, d), lambda i: (0, 0)),
        ],
        out_specs=[
            pl.BlockSpec((_TILE, _TOP_K), lambda i: (i, 0)),
            pl.BlockSpec((_TILE, _TOP_K), lambda i: (i, 0)),
        ],
        out_shape=[
            jax.ShapeDtypeStruct((n, _TOP_K), jnp.int32),
            jax.ShapeDtypeStruct((n, _TOP_K), jnp.float32),
        ],
        compiler_params=pltpu.CompilerParams(
            dimension_semantics=("parallel",)),
    )(x, W)
    return idx, gates
